# baseline (speedup 1.0000x reference)
import functools

import jax
import jax.numpy as jnp
from jax import lax
from jax.experimental import pallas as pl
from jax.experimental.pallas import tpu as pltpu
from jax.experimental.pallas import tpu_sc as plsc

_N = 10000
_E = 320000
_D = 128
_NP = 10240
_NC, _NS = 2, 16
_NW = _NC * _NS
_CH = 128
_CPW = 80
_EP = _NW * _CPW * _CH
_NBUF = 2
_RPS = _NP // _NS

_mesh = plsc.VectorSubcoreMesh(core_axis_name="c", subcore_axis_name="s")


@functools.partial(
    pl.kernel,
    out_type=(jax.ShapeDtypeStruct((_NC, _NP), jnp.float32),
              jax.ShapeDtypeStruct((_NC, _NP), jnp.float32)),
    mesh=_mesh,
    scratch_types=(
        pltpu.VMEM((_CPW * _CH,), jnp.int32),
        pltpu.VMEM((_CPW * _CH,), jnp.int32),
        tuple(pltpu.VMEM((_CH,), jnp.int32) for _ in range(2)),
        tuple(pltpu.VMEM((_CH,), jnp.int32) for _ in range(2)),
        tuple(pltpu.SemaphoreType.DMA for _ in range(2)),
        tuple(pltpu.SemaphoreType.DMA for _ in range(2)),
        pltpu.VMEM((_CH,), jnp.float32),
        pltpu.VMEM((_RPS,), jnp.float32),
        pltpu.VMEM_SHARED((_NP,), jnp.float32),
        pltpu.VMEM_SHARED((_NP,), jnp.float32),
    ),
)
def _degrees(src_hbm, dst_hbm, outs_hbm, outd_hbm,
             sidx, didx, sidx_b, didx_b, ssems, dsems, ones_v, zer_v,
             accs, accd):
    cid = lax.axis_index("c")
    sid = lax.axis_index("s")
    w = sid * _NC + cid
    one = jnp.full((16,), 1.0, jnp.float32)
    zero = jnp.zeros((16,), jnp.float32)
    for j in range(_CH // 16):
        ones_v[pl.ds(16 * j, 16)] = one
    for j in range(_RPS // 16):
        zer_v[pl.ds(16 * j, 16)] = zero
    base = sid * _RPS
    pltpu.sync_copy(zer_v, accs.at[pl.ds(base, _RPS)])
    pltpu.sync_copy(zer_v, accd.at[pl.ds(base, _RPS)])
    pltpu.sync_copy(src_hbm.at[pl.ds(w * _CPW * _CH, _CPW * _CH)], sidx)
    pltpu.sync_copy(dst_hbm.at[pl.ds(w * _CPW * _CH, _CPW * _CH)], didx)
    plsc.subcore_barrier()

    def fill_and_scat(j, p):
        for v in range(_CH // 16):
            sidx_b[p][pl.ds(16 * v, 16)] = sidx[pl.ds(j * _CH + 16 * v, 16)]
            didx_b[p][pl.ds(16 * v, 16)] = didx[pl.ds(j * _CH + 16 * v, 16)]
        pltpu.async_copy(ones_v, accs.at[sidx_b[p]], ssems[p], add=True)
        pltpu.async_copy(ones_v, accd.at[didx_b[p]], dsems[p], add=True)

    def ddrain(p):
        pltpu.make_async_copy(ones_v, accs.at[sidx_b[p]], ssems[p]).wait()
        pltpu.make_async_copy(ones_v, accd.at[didx_b[p]], dsems[p]).wait()

    for p in range(2):
        fill_and_scat(p, p)

    def body(m, carry):
        for p in range(2):
            ddrain(p)
            fill_and_scat(2 * m + 2 + p, p)
        return carry

    lax.fori_loop(0, _CPW // 2 - 1, body, 0)
    for p in range(2):
        ddrain(p)
    plsc.subcore_barrier()
    pltpu.sync_copy(accs.at[pl.ds(base, _RPS)],
                    outs_hbm.at[cid, pl.ds(base, _RPS)])
    pltpu.sync_copy(accd.at[pl.ds(base, _RPS)],
                    outd_hbm.at[cid, pl.ds(base, _RPS)])


@functools.partial(
    pl.kernel,
    out_type=jax.ShapeDtypeStruct((_NC, _NP, _D), jnp.float32),
    mesh=_mesh,
    scratch_types=(
        pltpu.VMEM((_CPW * _CH,), jnp.int32),
        tuple(pltpu.VMEM((_CH,), jnp.int32) for _ in range(_NBUF)),
        tuple(pltpu.VMEM((_CH, _D), jnp.float32) for _ in range(_NBUF)),
        tuple(pltpu.SemaphoreType.DMA for _ in range(_NBUF)),
        tuple(pltpu.SemaphoreType.DMA for _ in range(_NBUF)),
        tuple(pltpu.SemaphoreType.DMA for _ in range(_NBUF)),
        pltpu.SemaphoreType.DMA,
        pltpu.VMEM_SHARED((_NP, _D), jnp.float32),
    ),
)
def _aggregate(h_hbm, src_hbm, dst_hbm, out_hbm,
               sidx, didx_b, rows, gsems, isems, ssems, lsem, acc):
    cid = lax.axis_index("c")
    sid = lax.axis_index("s")
    w = sid * _NC + cid
    ebase = w * _CPW * _CH
    zero = jnp.zeros((16,), jnp.float32)

    ldesc = pltpu.async_copy(src_hbm.at[pl.ds(ebase, _CPW * _CH)], sidx, lsem)

    def zb(r, carry):
        for j in range(_D // 16):
            rows[0][r, pl.ds(16 * j, 16)] = zero
        return carry

    lax.fori_loop(0, _CH, zb, 0)
    rowbase = sid * _RPS

    def zc(k, carry):
        pltpu.sync_copy(rows[0], acc.at[pl.ds(rowbase + _CH * k, _CH)])
        return carry

    lax.fori_loop(0, _RPS // _CH, zc, 0)
    ldesc.wait()
    plsc.subcore_barrier()

    def drain(b):
        pltpu.make_async_copy(rows[b], acc.at[didx_b[b]], ssems[b]).wait()

    def start(k, b):
        pltpu.async_copy(dst_hbm.at[pl.ds(ebase + k * _CH, _CH)],
                         didx_b[b], isems[b])
        pltpu.async_copy(h_hbm.at[sidx.at[pl.ds(k * _CH, _CH)]],
                         rows[b], gsems[b])

    def finish(k, b):
        pltpu.make_async_copy(dst_hbm.at[pl.ds(ebase + k * _CH, _CH)],
                              didx_b[b], isems[b]).wait()
        pltpu.make_async_copy(h_hbm.at[sidx.at[pl.ds(k * _CH, _CH)]],
                              rows[b], gsems[b]).wait()
        pltpu.async_copy(rows[b], acc.at[didx_b[b]], ssems[b], add=True)

    for b in range(_NBUF):
        start(b, b)

    def body(j, carry):
        for b in range(_NBUF):
            finish(_NBUF * j + b, b)
        for b in range(_NBUF):
            drain(b)
            start(_NBUF * j + _NBUF + b, b)
        return carry

    lax.fori_loop(0, _CPW // _NBUF - 1, body, 0)
    for b in range(_NBUF):
        finish(_CPW - _NBUF + b, b)
    for b in range(_NBUF):
        drain(b)

    plsc.subcore_barrier()
    pltpu.sync_copy(acc.at[pl.ds(rowbase, _RPS)],
                    out_hbm.at[cid, pl.ds(rowbase, _RPS)])


_R = 1024
_G = _NP // _R


def _t0_body(x_ref, s0_ref, s1_ref, o_ref):
    ns = lax.rsqrt(jnp.maximum(s0_ref[...] + s1_ref[...], 1.0))
    o_ref[...] = x_ref[...] * ns


_t0 = pl.pallas_call(
    _t0_body,
    grid=(_G,),
    in_specs=[
        pl.BlockSpec((_R, _D), lambda i: (i, 0)),
        pl.BlockSpec((_R, 1), lambda i: (i, 0)),
        pl.BlockSpec((_R, 1), lambda i: (i, 0)),
    ],
    out_specs=pl.BlockSpec((_R, _D), lambda i: (i, 0)),
    out_shape=jax.ShapeDtypeStruct((_NP, _D), jnp.float32),
)


def _tmid_body(agg_ref, d0_ref, d1_ref, s0_ref, s1_ref, b_ref, a_ref, w_ref,
               o_ref):
    h = jnp.dot(agg_ref[0] + agg_ref[1], w_ref[...],
                preferred_element_type=jnp.float32)
    nd = lax.rsqrt(jnp.maximum(d0_ref[...] + d1_ref[...], 1.0))
    h = h * nd + b_ref[...]
    h = jnp.where(h >= 0, h, a_ref[...] * h)
    ns = lax.rsqrt(jnp.maximum(s0_ref[...] + s1_ref[...], 1.0))
    o_ref[...] = h * ns


_tmid = pl.pallas_call(
    _tmid_body,
    grid=(_G,),
    in_specs=[
        pl.BlockSpec((_NC, _R, _D), lambda i: (0, i, 0)),
        pl.BlockSpec((_R, 1), lambda i: (i, 0)),
        pl.BlockSpec((_R, 1), lambda i: (i, 0)),
        pl.BlockSpec((_R, 1), lambda i: (i, 0)),
        pl.BlockSpec((_R, 1), lambda i: (i, 0)),
        pl.BlockSpec((1, _D), lambda i: (0, 0)),
        pl.BlockSpec((1, _D), lambda i: (0, 0)),
        pl.BlockSpec((_D, _D), lambda i: (0, 0)),
    ],
    out_specs=pl.BlockSpec((_R, _D), lambda i: (i, 0)),
    out_shape=jax.ShapeDtypeStruct((_NP, _D), jnp.float32),
)


def _t3_body(agg_ref, d0_ref, d1_ref, b_ref, w_ref, o_ref):
    h = jnp.dot(agg_ref[0] + agg_ref[1], w_ref[...],
                preferred_element_type=jnp.float32)
    nd = lax.rsqrt(jnp.maximum(d0_ref[...] + d1_ref[...], 1.0))
    o_ref[...] = h * nd + b_ref[...]


_t3 = pl.pallas_call(
    _t3_body,
    grid=(_G,),
    in_specs=[
        pl.BlockSpec((_NC, _R, _D), lambda i: (0, i, 0)),
        pl.BlockSpec((_R, 1), lambda i: (i, 0)),
        pl.BlockSpec((_R, 1), lambda i: (i, 0)),
        pl.BlockSpec((1, _D), lambda i: (0, 0)),
        pl.BlockSpec((_D, _D), lambda i: (0, 0)),
    ],
    out_specs=pl.BlockSpec((_R, _D), lambda i: (i, 0)),
    out_shape=jax.ShapeDtypeStruct((_NP, _D), jnp.float32),
)


def kernel(feat, edge_index, W1, b1, a1, W2, b2, a2, W3, b3):
    src = edge_index[0]
    dst = edge_index[1]
    pad = _N + (jnp.arange(_EP - _E, dtype=jnp.int32) % (_NP - _N))
    srcp = jnp.concatenate([src, pad])
    dstp = jnp.concatenate([dst, pad])

    degS, degD = _degrees(srcp, dstp)
    s0 = degS[0].reshape(_NP, 1)
    s1 = degS[1].reshape(_NP, 1)
    d0 = degD[0].reshape(_NP, 1)
    d1 = degD[1].reshape(_NP, 1)
    xp = jnp.pad(feat, ((0, _NP - _N), (0, 0)))
    b1r, a1r = b1.reshape(1, _D), a1.reshape(1, _D)
    b2r, a2r = b2.reshape(1, _D), a2.reshape(1, _D)
    b3r = b3.reshape(1, _D)

    q = _t0(xp, s0, s1)
    agg = _aggregate(q, srcp, dstp)
    q = _tmid(agg, d0, d1, s0, s1, b1r, a1r, W1)
    agg = _aggregate(q, srcp, dstp)
    q = _tmid(agg, d0, d1, s0, s1, b2r, a2r, W2)
    agg = _aggregate(q, srcp, dstp)
    out = _t3(agg, d0, d1, b3r, W3)
    return out[:_N]


# speedup vs baseline: 1.2695x; 1.2695x over previous
import functools

import jax
import jax.numpy as jnp
from jax import lax
from jax.experimental import pallas as pl
from jax.experimental.pallas import tpu as pltpu
from jax.experimental.pallas import tpu_sc as plsc

_N = 10000
_E = 320000
_D = 128
_NP = 10240
_NC, _NS = 2, 16
_NW = _NC * _NS
_CH = 128
_CPW = 80
_EP = _NW * _CPW * _CH
_NBUF = 2
_RPS = _NP // _NS

_mesh = plsc.VectorSubcoreMesh(core_axis_name="c", subcore_axis_name="s")


@functools.partial(
    pl.kernel,
    out_type=(jax.ShapeDtypeStruct((_NC, _NP), jnp.float32),
              jax.ShapeDtypeStruct((_NC, _NP), jnp.float32)),
    mesh=_mesh,
    scratch_types=(
        pltpu.VMEM((_CPW * _CH,), jnp.int32),
        pltpu.VMEM((_CPW * _CH,), jnp.int32),
        tuple(pltpu.VMEM((_CH,), jnp.int32) for _ in range(2)),
        tuple(pltpu.VMEM((_CH,), jnp.int32) for _ in range(2)),
        tuple(pltpu.SemaphoreType.DMA for _ in range(2)),
        tuple(pltpu.SemaphoreType.DMA for _ in range(2)),
        pltpu.VMEM((_CH,), jnp.float32),
        pltpu.VMEM((_RPS,), jnp.float32),
        pltpu.VMEM_SHARED((_NP,), jnp.float32),
        pltpu.VMEM_SHARED((_NP,), jnp.float32),
    ),
)
def _degrees(src_hbm, dst_hbm, outs_hbm, outd_hbm,
             sidx, didx, sidx_b, didx_b, ssems, dsems, ones_v, zer_v,
             accs, accd):
    cid = lax.axis_index("c")
    sid = lax.axis_index("s")
    w = sid * _NC + cid
    one = jnp.full((16,), 1.0, jnp.float32)
    zero = jnp.zeros((16,), jnp.float32)
    for j in range(_CH // 16):
        ones_v[pl.ds(16 * j, 16)] = one
    for j in range(_RPS // 16):
        zer_v[pl.ds(16 * j, 16)] = zero
    base = sid * _RPS
    pltpu.sync_copy(zer_v, accs.at[pl.ds(base, _RPS)])
    pltpu.sync_copy(zer_v, accd.at[pl.ds(base, _RPS)])
    pltpu.sync_copy(src_hbm.at[pl.ds(w * _CPW * _CH, _CPW * _CH)], sidx)
    pltpu.sync_copy(dst_hbm.at[pl.ds(w * _CPW * _CH, _CPW * _CH)], didx)
    plsc.subcore_barrier()

    def fill_and_scat(j, p):
        for v in range(_CH // 16):
            sidx_b[p][pl.ds(16 * v, 16)] = sidx[pl.ds(j * _CH + 16 * v, 16)]
            didx_b[p][pl.ds(16 * v, 16)] = didx[pl.ds(j * _CH + 16 * v, 16)]
        pltpu.async_copy(ones_v, accs.at[sidx_b[p]], ssems[p], add=True)
        pltpu.async_copy(ones_v, accd.at[didx_b[p]], dsems[p], add=True)

    def ddrain(p):
        pltpu.make_async_copy(ones_v, accs.at[sidx_b[p]], ssems[p]).wait()
        pltpu.make_async_copy(ones_v, accd.at[didx_b[p]], dsems[p]).wait()

    for p in range(2):
        fill_and_scat(p, p)

    def body(m, carry):
        for p in range(2):
            ddrain(p)
            fill_and_scat(2 * m + 2 + p, p)
        return carry

    lax.fori_loop(0, _CPW // 2 - 1, body, 0)
    for p in range(2):
        ddrain(p)
    plsc.subcore_barrier()
    pltpu.sync_copy(accs.at[pl.ds(base, _RPS)],
                    outs_hbm.at[cid, pl.ds(base, _RPS)])
    pltpu.sync_copy(accd.at[pl.ds(base, _RPS)],
                    outd_hbm.at[cid, pl.ds(base, _RPS)])


@functools.partial(
    pl.kernel,
    out_type=jax.ShapeDtypeStruct((_NC, _NP, _D), jnp.float32),
    mesh=_mesh,
    scratch_types=(
        pltpu.VMEM((_CPW * _CH,), jnp.int32),
        tuple(pltpu.VMEM((_CH,), jnp.int32) for _ in range(_NBUF)),
        tuple(pltpu.VMEM((_CH, _D), jnp.float32) for _ in range(_NBUF)),
        tuple(pltpu.SemaphoreType.DMA for _ in range(_NBUF)),
        tuple(pltpu.SemaphoreType.DMA for _ in range(_NBUF)),
        pltpu.SemaphoreType.DMA,
        pltpu.VMEM_SHARED((_NP, _D), jnp.float32),
    ),
)
def _aggregate(h_hbm, src_hbm, dst_hbm, out_hbm,
               sidx, didx_b, rows, gsems, isems, lsem, acc):
    cid = lax.axis_index("c")
    sid = lax.axis_index("s")
    w = sid * _NC + cid
    ebase = w * _CPW * _CH
    zero = jnp.zeros((16,), jnp.float32)

    ldesc = pltpu.async_copy(src_hbm.at[pl.ds(ebase, _CPW * _CH)], sidx, lsem)

    def zb(r, carry):
        for j in range(_D // 16):
            rows[0][r, pl.ds(16 * j, 16)] = zero
        return carry

    lax.fori_loop(0, _CH, zb, 0)
    rowbase = sid * _RPS

    def zc(k, carry):
        pltpu.sync_copy(rows[0], acc.at[pl.ds(rowbase + _CH * k, _CH)])
        return carry

    lax.fori_loop(0, _RPS // _CH, zc, 0)
    ldesc.wait()
    plsc.subcore_barrier()

    def start(k, b):
        pltpu.async_copy(dst_hbm.at[pl.ds(ebase + k * _CH, _CH)],
                         didx_b[b], isems[b])
        pltpu.async_copy(h_hbm.at[sidx.at[pl.ds(k * _CH, _CH)]],
                         rows[b], gsems[b])

    def finish(k, b):
        pltpu.make_async_copy(dst_hbm.at[pl.ds(ebase + k * _CH, _CH)],
                              didx_b[b], isems[b]).wait()
        pltpu.make_async_copy(h_hbm.at[sidx.at[pl.ds(k * _CH, _CH)]],
                              rows[b], gsems[b]).wait()
        pltpu.sync_copy(rows[b], acc.at[didx_b[b]], add=True)

    for b in range(_NBUF):
        start(b, b)

    def body(j, carry):
        for b in range(_NBUF):
            k = _NBUF * j + b
            finish(k, b)
            start(k + _NBUF, b)
        return carry

    lax.fori_loop(0, _CPW // _NBUF - 1, body, 0)
    for b in range(_NBUF):
        finish(_CPW - _NBUF + b, b)

    plsc.subcore_barrier()
    pltpu.sync_copy(acc.at[pl.ds(rowbase, _RPS)],
                    out_hbm.at[cid, pl.ds(rowbase, _RPS)])


_R = 1024
_G = _NP // _R


def _t0_body(x_ref, s0_ref, s1_ref, o_ref):
    ns = lax.rsqrt(jnp.maximum(s0_ref[...] + s1_ref[...], 1.0))
    o_ref[...] = x_ref[...] * ns


_t0 = pl.pallas_call(
    _t0_body,
    grid=(_G,),
    in_specs=[
        pl.BlockSpec((_R, _D), lambda i: (i, 0)),
        pl.BlockSpec((_R, 1), lambda i: (i, 0)),
        pl.BlockSpec((_R, 1), lambda i: (i, 0)),
    ],
    out_specs=pl.BlockSpec((_R, _D), lambda i: (i, 0)),
    out_shape=jax.ShapeDtypeStruct((_NP, _D), jnp.float32),
)


def _tmid_body(agg_ref, d0_ref, d1_ref, s0_ref, s1_ref, b_ref, a_ref, w_ref,
               o_ref):
    h = jnp.dot(agg_ref[0] + agg_ref[1], w_ref[...],
                preferred_element_type=jnp.float32)
    nd = lax.rsqrt(jnp.maximum(d0_ref[...] + d1_ref[...], 1.0))
    h = h * nd + b_ref[...]
    h = jnp.where(h >= 0, h, a_ref[...] * h)
    ns = lax.rsqrt(jnp.maximum(s0_ref[...] + s1_ref[...], 1.0))
    o_ref[...] = h * ns


_tmid = pl.pallas_call(
    _tmid_body,
    grid=(_G,),
    in_specs=[
        pl.BlockSpec((_NC, _R, _D), lambda i: (0, i, 0)),
        pl.BlockSpec((_R, 1), lambda i: (i, 0)),
        pl.BlockSpec((_R, 1), lambda i: (i, 0)),
        pl.BlockSpec((_R, 1), lambda i: (i, 0)),
        pl.BlockSpec((_R, 1), lambda i: (i, 0)),
        pl.BlockSpec((1, _D), lambda i: (0, 0)),
        pl.BlockSpec((1, _D), lambda i: (0, 0)),
        pl.BlockSpec((_D, _D), lambda i: (0, 0)),
    ],
    out_specs=pl.BlockSpec((_R, _D), lambda i: (i, 0)),
    out_shape=jax.ShapeDtypeStruct((_NP, _D), jnp.float32),
)


def _t3_body(agg_ref, d0_ref, d1_ref, b_ref, w_ref, o_ref):
    h = jnp.dot(agg_ref[0] + agg_ref[1], w_ref[...],
                preferred_element_type=jnp.float32)
    nd = lax.rsqrt(jnp.maximum(d0_ref[...] + d1_ref[...], 1.0))
    o_ref[...] = h * nd + b_ref[...]


_t3 = pl.pallas_call(
    _t3_body,
    grid=(_G,),
    in_specs=[
        pl.BlockSpec((_NC, _R, _D), lambda i: (0, i, 0)),
        pl.BlockSpec((_R, 1), lambda i: (i, 0)),
        pl.BlockSpec((_R, 1), lambda i: (i, 0)),
        pl.BlockSpec((1, _D), lambda i: (0, 0)),
        pl.BlockSpec((_D, _D), lambda i: (0, 0)),
    ],
    out_specs=pl.BlockSpec((_R, _D), lambda i: (i, 0)),
    out_shape=jax.ShapeDtypeStruct((_NP, _D), jnp.float32),
)


def kernel(feat, edge_index, W1, b1, a1, W2, b2, a2, W3, b3):
    src = edge_index[0]
    dst = edge_index[1]
    pad = _N + (jnp.arange(_EP - _E, dtype=jnp.int32) % (_NP - _N))
    srcp = jnp.concatenate([src, pad])
    dstp = jnp.concatenate([dst, pad])

    degS, degD = _degrees(srcp, dstp)
    s0 = degS[0].reshape(_NP, 1)
    s1 = degS[1].reshape(_NP, 1)
    d0 = degD[0].reshape(_NP, 1)
    d1 = degD[1].reshape(_NP, 1)
    xp = jnp.pad(feat, ((0, _NP - _N), (0, 0)))
    b1r, a1r = b1.reshape(1, _D), a1.reshape(1, _D)
    b2r, a2r = b2.reshape(1, _D), a2.reshape(1, _D)
    b3r = b3.reshape(1, _D)

    q = _t0(xp, s0, s1)
    agg = _aggregate(q, srcp, dstp)
    q = _tmid(agg, d0, d1, s0, s1, b1r, a1r, W1)
    agg = _aggregate(q, srcp, dstp)
    q = _tmid(agg, d0, d1, s0, s1, b2r, a2r, W2)
    agg = _aggregate(q, srcp, dstp)
    out = _t3(agg, d0, d1, b3r, W3)
    return out[:_N]


# speedup vs baseline: 1.2805x; 1.0087x over previous
import functools

import jax
import jax.numpy as jnp
from jax import lax
from jax.experimental import pallas as pl
from jax.experimental.pallas import tpu as pltpu
from jax.experimental.pallas import tpu_sc as plsc

_N = 10000
_E = 320000
_D = 128
_NP = 10240
_NC, _NS = 2, 16
_NW = _NC * _NS
_EW = _E // _NW
_CH = 128
_NF = _EW // _CH
_TAIL = _EW - _NF * _CH
_NBUF = 2
_RPS = _NP // _NS
_DPS = 2 * _NP // _NS

_mesh = plsc.VectorSubcoreMesh(core_axis_name="c", subcore_axis_name="s")


@functools.partial(
    pl.kernel,
    out_type=jax.ShapeDtypeStruct((_NC, 2 * _NP), jnp.float32),
    mesh=_mesh,
    scratch_types=(
        pltpu.VMEM((_EW,), jnp.int32),
        pltpu.VMEM((_EW,), jnp.int32),
        tuple(pltpu.VMEM((_CH,), jnp.int32) for _ in range(2)),
        tuple(pltpu.VMEM((_CH,), jnp.int32) for _ in range(2)),
        pltpu.VMEM((_TAIL,), jnp.int32),
        pltpu.VMEM((_TAIL,), jnp.int32),
        tuple(pltpu.SemaphoreType.DMA for _ in range(2)),
        tuple(pltpu.SemaphoreType.DMA for _ in range(2)),
        pltpu.VMEM((_CH,), jnp.float32),
        pltpu.VMEM((_DPS,), jnp.float32),
        pltpu.VMEM_SHARED((2 * _NP,), jnp.float32),
    ),
)
def _degrees(src_hbm, dst_hbm, out_hbm,
             sidx, didx, sidx_b, didx_b, sidx_t, didx_t, ssems, dsems,
             ones_v, zer_v, acc):
    cid = lax.axis_index("c")
    sid = lax.axis_index("s")
    w = sid * _NC + cid
    one = jnp.full((16,), 1.0, jnp.float32)
    zero = jnp.zeros((16,), jnp.float32)
    for j in range(_CH // 16):
        ones_v[pl.ds(16 * j, 16)] = one
    for j in range(_DPS // 16):
        zer_v[pl.ds(16 * j, 16)] = zero
    base = sid * _DPS
    pltpu.sync_copy(zer_v, acc.at[pl.ds(base, _DPS)])
    pltpu.sync_copy(src_hbm.at[pl.ds(w * _EW, _EW)], sidx)
    pltpu.sync_copy(dst_hbm.at[pl.ds(w * _EW, _EW)], didx)
    plsc.subcore_barrier()

    def fill_and_scat(j, p):
        for v in range(_CH // 16):
            sl = pl.ds(j * _CH + 16 * v, 16)
            sidx_b[p][pl.ds(16 * v, 16)] = sidx[sl] * 2
            didx_b[p][pl.ds(16 * v, 16)] = didx[sl] * 2 + 1
        pltpu.async_copy(ones_v, acc.at[sidx_b[p]], ssems[p], add=True)
        pltpu.async_copy(ones_v, acc.at[didx_b[p]], dsems[p], add=True)

    def ddrain(p):
        pltpu.make_async_copy(ones_v, acc.at[sidx_b[p]], ssems[p]).wait()
        pltpu.make_async_copy(ones_v, acc.at[didx_b[p]], dsems[p]).wait()

    for p in range(2):
        fill_and_scat(p, p)

    def body(m, carry):
        for p in range(2):
            ddrain(p)
            fill_and_scat(2 * m + 2 + p, p)
        return carry

    lax.fori_loop(0, _NF // 2 - 1, body, 0)
    for p in range(2):
        ddrain(p)

    sidx_t[...] = sidx[pl.ds(_NF * _CH, _TAIL)] * 2
    didx_t[...] = didx[pl.ds(_NF * _CH, _TAIL)] * 2 + 1
    pltpu.sync_copy(ones_v.at[pl.ds(0, _TAIL)], acc.at[sidx_t], add=True)
    pltpu.sync_copy(ones_v.at[pl.ds(0, _TAIL)], acc.at[didx_t], add=True)

    plsc.subcore_barrier()
    pltpu.sync_copy(acc.at[pl.ds(base, _DPS)],
                    out_hbm.at[cid, pl.ds(base, _DPS)])


@functools.partial(
    pl.kernel,
    out_type=jax.ShapeDtypeStruct((_NC, _NP, _D), jnp.float32),
    mesh=_mesh,
    scratch_types=(
        pltpu.VMEM((_EW,), jnp.int32),
        tuple(pltpu.VMEM((_CH,), jnp.int32) for _ in range(_NBUF)),
        pltpu.VMEM((_TAIL,), jnp.int32),
        tuple(pltpu.VMEM((_CH, _D), jnp.float32) for _ in range(_NBUF)),
        tuple(pltpu.SemaphoreType.DMA for _ in range(_NBUF)),
        tuple(pltpu.SemaphoreType.DMA for _ in range(_NBUF)),
        pltpu.SemaphoreType.DMA,
        pltpu.VMEM_SHARED((_NP, _D), jnp.float32),
    ),
)
def _aggregate(h_hbm, src_hbm, dst_hbm, out_hbm,
               sidx, didx_b, didx_t, rows, gsems, isems, lsem, acc):
    cid = lax.axis_index("c")
    sid = lax.axis_index("s")
    w = sid * _NC + cid
    ebase = w * _EW
    zero = jnp.zeros((16,), jnp.float32)

    ldesc = pltpu.async_copy(src_hbm.at[pl.ds(ebase, _EW)], sidx, lsem)

    def zb(r, carry):
        for j in range(_D // 16):
            rows[0][r, pl.ds(16 * j, 16)] = zero
        return carry

    lax.fori_loop(0, _CH, zb, 0)
    rowbase = sid * _RPS

    def zc(k, carry):
        pltpu.sync_copy(rows[0], acc.at[pl.ds(rowbase + _CH * k, _CH)])
        return carry

    lax.fori_loop(0, _RPS // _CH, zc, 0)
    ldesc.wait()
    plsc.subcore_barrier()

    def start(k, b):
        pltpu.async_copy(dst_hbm.at[pl.ds(ebase + k * _CH, _CH)],
                         didx_b[b], isems[b])
        pltpu.async_copy(h_hbm.at[sidx.at[pl.ds(k * _CH, _CH)]],
                         rows[b], gsems[b])

    def finish(k, b):
        pltpu.make_async_copy(dst_hbm.at[pl.ds(ebase + k * _CH, _CH)],
                              didx_b[b], isems[b]).wait()
        pltpu.make_async_copy(h_hbm.at[sidx.at[pl.ds(k * _CH, _CH)]],
                              rows[b], gsems[b]).wait()
        pltpu.sync_copy(rows[b], acc.at[didx_b[b]], add=True)

    for b in range(_NBUF):
        start(b, b)

    def body(j, carry):
        for b in range(_NBUF):
            k = _NBUF * j + b
            finish(k, b)
            start(k + _NBUF, b)
        return carry

    lax.fori_loop(0, _NF // _NBUF - 1, body, 0)
    for b in range(_NBUF):
        finish(_NF - _NBUF + b, b)

    pltpu.sync_copy(dst_hbm.at[pl.ds(ebase + _NF * _CH, _TAIL)], didx_t)
    pltpu.sync_copy(h_hbm.at[sidx.at[pl.ds(_NF * _CH, _TAIL)]],
                    rows[0].at[pl.ds(0, _TAIL)])
    pltpu.sync_copy(rows[0].at[pl.ds(0, _TAIL)], acc.at[didx_t], add=True)

    plsc.subcore_barrier()
    pltpu.sync_copy(acc.at[pl.ds(rowbase, _RPS)],
                    out_hbm.at[cid, pl.ds(rowbase, _RPS)])


_R = 1000
_G = _N // _R


def _ns(sd):
    return lax.rsqrt(jnp.maximum(sd[:, 0:1], 1.0))


def _nd(sd):
    return lax.rsqrt(jnp.maximum(sd[:, 1:2], 1.0))


def _t0_body(x_ref, sd_ref, o_ref):
    sd = sd_ref[0] + sd_ref[1]
    o_ref[...] = x_ref[...] * _ns(sd)


_t0 = pl.pallas_call(
    _t0_body,
    grid=(_G,),
    in_specs=[
        pl.BlockSpec((_R, _D), lambda i: (i, 0)),
        pl.BlockSpec((_NC, _R, 2), lambda i: (0, i, 0)),
    ],
    out_specs=pl.BlockSpec((_R, _D), lambda i: (i, 0)),
    out_shape=jax.ShapeDtypeStruct((_N, _D), jnp.float32),
)


def _tmid_body(agg_ref, sd_ref, b_ref, a_ref, w_ref, o_ref):
    h = jnp.dot(agg_ref[0] + agg_ref[1], w_ref[...],
                preferred_element_type=jnp.float32)
    sd = sd_ref[0] + sd_ref[1]
    h = h * _nd(sd) + b_ref[...]
    h = jnp.where(h >= 0, h, a_ref[...] * h)
    o_ref[...] = h * _ns(sd)


_tmid = pl.pallas_call(
    _tmid_body,
    grid=(_G,),
    in_specs=[
        pl.BlockSpec((_NC, _R, _D), lambda i: (0, i, 0)),
        pl.BlockSpec((_NC, _R, 2), lambda i: (0, i, 0)),
        pl.BlockSpec((1, _D), lambda i: (0, 0)),
        pl.BlockSpec((1, _D), lambda i: (0, 0)),
        pl.BlockSpec((_D, _D), lambda i: (0, 0)),
    ],
    out_specs=pl.BlockSpec((_R, _D), lambda i: (i, 0)),
    out_shape=jax.ShapeDtypeStruct((_N, _D), jnp.float32),
)


def _t3_body(agg_ref, sd_ref, b_ref, w_ref, o_ref):
    h = jnp.dot(agg_ref[0] + agg_ref[1], w_ref[...],
                preferred_element_type=jnp.float32)
    sd = sd_ref[0] + sd_ref[1]
    o_ref[...] = h * _nd(sd) + b_ref[...]


_t3 = pl.pallas_call(
    _t3_body,
    grid=(_G,),
    in_specs=[
        pl.BlockSpec((_NC, _R, _D), lambda i: (0, i, 0)),
        pl.BlockSpec((_NC, _R, 2), lambda i: (0, i, 0)),
        pl.BlockSpec((1, _D), lambda i: (0, 0)),
        pl.BlockSpec((_D, _D), lambda i: (0, 0)),
    ],
    out_specs=pl.BlockSpec((_R, _D), lambda i: (i, 0)),
    out_shape=jax.ShapeDtypeStruct((_N, _D), jnp.float32),
)


def kernel(feat, edge_index, W1, b1, a1, W2, b2, a2, W3, b3):
    src = edge_index[0]
    dst = edge_index[1]

    degsd = _degrees(src, dst)
    sd = degsd.reshape(_NC, _NP, 2)
    b1r, a1r = b1.reshape(1, _D), a1.reshape(1, _D)
    b2r, a2r = b2.reshape(1, _D), a2.reshape(1, _D)
    b3r = b3.reshape(1, _D)

    q = _t0(feat, sd)
    agg = _aggregate(q, src, dst)
    q = _tmid(agg, sd, b1r, a1r, W1)
    agg = _aggregate(q, src, dst)
    q = _tmid(agg, sd, b2r, a2r, W2)
    agg = _aggregate(q, src, dst)
    return _t3(agg, sd, b3r, W3)


# speedup vs baseline: 1.3298x; 1.0385x over previous
import functools

import jax
import jax.numpy as jnp
from jax import lax
from jax.experimental import pallas as pl
from jax.experimental.pallas import tpu as pltpu
from jax.experimental.pallas import tpu_sc as plsc

_N = 10000
_E = 320000
_D = 128
_NP = 10240
_NC, _NS = 2, 16
_NW = _NC * _NS
_EW = _E // _NW
_CH = 128
_NF = _EW // _CH
_TAIL = _EW - _NF * _CH
_NBUF = 2
_RPS = _NP // _NS
_DPS = _NP // _NS

_mesh = plsc.VectorSubcoreMesh(core_axis_name="c", subcore_axis_name="s")


@functools.partial(
    pl.kernel,
    out_type=(jax.ShapeDtypeStruct((_NC, _NP), jnp.float32),
              jax.ShapeDtypeStruct((_NC, _NP), jnp.float32)),
    mesh=_mesh,
    scratch_types=(
        pltpu.VMEM((_EW,), jnp.int32),
        pltpu.VMEM((_EW,), jnp.int32),
        tuple(pltpu.VMEM((_CH,), jnp.int32) for _ in range(2)),
        tuple(pltpu.VMEM((_CH,), jnp.int32) for _ in range(2)),
        pltpu.VMEM((_TAIL,), jnp.int32),
        pltpu.VMEM((_TAIL,), jnp.int32),
        tuple(pltpu.SemaphoreType.DMA for _ in range(2)),
        tuple(pltpu.SemaphoreType.DMA for _ in range(2)),
        pltpu.VMEM((_CH,), jnp.float32),
        pltpu.VMEM((_DPS,), jnp.float32),
        pltpu.VMEM_SHARED((_NP,), jnp.float32),
        pltpu.VMEM_SHARED((_NP,), jnp.float32),
    ),
)
def _degrees(edge_hbm, outs_hbm, outd_hbm,
             sidx, didx, sidx_b, didx_b, sidx_t, didx_t, ssems, dsems,
             ones_v, zer_v, accs, accd):
    cid = lax.axis_index("c")
    sid = lax.axis_index("s")
    w = sid * _NC + cid
    one = jnp.full((16,), 1.0, jnp.float32)
    zero = jnp.zeros((16,), jnp.float32)
    for j in range(_CH // 16):
        ones_v[pl.ds(16 * j, 16)] = one
    for j in range(_DPS // 16):
        zer_v[pl.ds(16 * j, 16)] = zero
    base = sid * _DPS
    pltpu.sync_copy(zer_v, accs.at[pl.ds(base, _DPS)])
    pltpu.sync_copy(zer_v, accd.at[pl.ds(base, _DPS)])
    pltpu.sync_copy(edge_hbm.at[pl.ds(w * _EW, _EW)], sidx)
    pltpu.sync_copy(edge_hbm.at[pl.ds(_E + w * _EW, _EW)], didx)
    plsc.subcore_barrier()

    def fill_and_scat(j, p):
        for v in range(_CH // 16):
            sl = pl.ds(j * _CH + 16 * v, 16)
            sidx_b[p][pl.ds(16 * v, 16)] = sidx[sl]
            didx_b[p][pl.ds(16 * v, 16)] = didx[sl]
        pltpu.async_copy(ones_v, accs.at[sidx_b[p]], ssems[p], add=True)
        pltpu.async_copy(ones_v, accd.at[didx_b[p]], dsems[p], add=True)

    def ddrain(p):
        pltpu.make_async_copy(ones_v, accs.at[sidx_b[p]], ssems[p]).wait()
        pltpu.make_async_copy(ones_v, accd.at[didx_b[p]], dsems[p]).wait()

    for p in range(2):
        fill_and_scat(p, p)

    def body(m, carry):
        for p in range(2):
            ddrain(p)
            fill_and_scat(2 * m + 2 + p, p)
        return carry

    lax.fori_loop(0, _NF // 2 - 1, body, 0)
    for p in range(2):
        ddrain(p)

    sidx_t[...] = sidx[pl.ds(_NF * _CH, _TAIL)]
    didx_t[...] = didx[pl.ds(_NF * _CH, _TAIL)]
    pltpu.sync_copy(ones_v.at[pl.ds(0, _TAIL)], accs.at[sidx_t], add=True)
    pltpu.sync_copy(ones_v.at[pl.ds(0, _TAIL)], accd.at[didx_t], add=True)

    plsc.subcore_barrier()
    pltpu.sync_copy(accs.at[pl.ds(base, _DPS)],
                    outs_hbm.at[cid, pl.ds(base, _DPS)])
    pltpu.sync_copy(accd.at[pl.ds(base, _DPS)],
                    outd_hbm.at[cid, pl.ds(base, _DPS)])


@functools.partial(
    pl.kernel,
    out_type=jax.ShapeDtypeStruct((_NC, _NP, _D), jnp.float32),
    mesh=_mesh,
    scratch_types=(
        pltpu.VMEM((_EW,), jnp.int32),
        tuple(pltpu.VMEM((_CH,), jnp.int32) for _ in range(_NBUF)),
        pltpu.VMEM((_TAIL,), jnp.int32),
        tuple(pltpu.VMEM((_CH, _D), jnp.float32) for _ in range(_NBUF)),
        tuple(pltpu.SemaphoreType.DMA for _ in range(_NBUF)),
        tuple(pltpu.SemaphoreType.DMA for _ in range(_NBUF)),
        pltpu.SemaphoreType.DMA,
        pltpu.SemaphoreType.DMA,
        pltpu.VMEM_SHARED((_NP, _D), jnp.float32),
    ),
)
def _aggregate(h_hbm, edge_hbm, zeros_hbm, out_hbm,
               sidx, didx_b, didx_t, rows, gsems, isems, lsem, zsem, acc):
    cid = lax.axis_index("c")
    sid = lax.axis_index("s")
    w = sid * _NC + cid
    ebase = w * _EW
    rowbase = sid * _RPS

    zdesc = pltpu.async_copy(zeros_hbm.at[pl.ds(rowbase, _RPS)],
                             acc.at[pl.ds(rowbase, _RPS)], zsem)
    ldesc = pltpu.async_copy(edge_hbm.at[pl.ds(ebase, _EW)], sidx, lsem)
    ldesc.wait()

    def start(k, b):
        pltpu.async_copy(edge_hbm.at[pl.ds(_E + ebase + k * _CH, _CH)],
                         didx_b[b], isems[b])
        pltpu.async_copy(h_hbm.at[sidx.at[pl.ds(k * _CH, _CH)]],
                         rows[b], gsems[b])

    def finish(k, b):
        pltpu.make_async_copy(edge_hbm.at[pl.ds(_E + ebase + k * _CH, _CH)],
                              didx_b[b], isems[b]).wait()
        pltpu.make_async_copy(h_hbm.at[sidx.at[pl.ds(k * _CH, _CH)]],
                              rows[b], gsems[b]).wait()
        pltpu.sync_copy(rows[b], acc.at[didx_b[b]], add=True)

    for b in range(_NBUF):
        start(b, b)

    zdesc.wait()
    plsc.subcore_barrier()

    def body(j, carry):
        for b in range(_NBUF):
            k = _NBUF * j + b
            finish(k, b)
            start(k + _NBUF, b)
        return carry

    lax.fori_loop(0, _NF // _NBUF - 1, body, 0)
    for b in range(_NBUF):
        finish(_NF - _NBUF + b, b)

    pltpu.sync_copy(edge_hbm.at[pl.ds(_E + ebase + _NF * _CH, _TAIL)], didx_t)
    pltpu.sync_copy(h_hbm.at[sidx.at[pl.ds(_NF * _CH, _TAIL)]],
                    rows[0].at[pl.ds(0, _TAIL)])
    pltpu.sync_copy(rows[0].at[pl.ds(0, _TAIL)], acc.at[didx_t], add=True)

    plsc.subcore_barrier()
    pltpu.sync_copy(acc.at[pl.ds(rowbase, _RPS)],
                    out_hbm.at[cid, pl.ds(rowbase, _RPS)])


_R = 2000
_G = _N // _R


def _t0_body(x_ref, s0_ref, s1_ref, o_ref):
    ns = lax.rsqrt(jnp.maximum(s0_ref[...] + s1_ref[...], 1.0))
    o_ref[...] = x_ref[...] * ns


_t0 = pl.pallas_call(
    _t0_body,
    grid=(_G,),
    in_specs=[
        pl.BlockSpec((_R, _D), lambda i: (i, 0)),
        pl.BlockSpec((_R, 1), lambda i: (i, 0)),
        pl.BlockSpec((_R, 1), lambda i: (i, 0)),
    ],
    out_specs=pl.BlockSpec((_R, _D), lambda i: (i, 0)),
    out_shape=jax.ShapeDtypeStruct((_N, _D), jnp.float32),
)


def _tmid_body(agg_ref, d0_ref, d1_ref, s0_ref, s1_ref, b_ref, a_ref, w_ref,
               o_ref):
    h = jnp.dot(agg_ref[0] + agg_ref[1], w_ref[...],
                preferred_element_type=jnp.float32)
    nd = lax.rsqrt(jnp.maximum(d0_ref[...] + d1_ref[...], 1.0))
    h = h * nd + b_ref[...]
    h = jnp.where(h >= 0, h, a_ref[...] * h)
    ns = lax.rsqrt(jnp.maximum(s0_ref[...] + s1_ref[...], 1.0))
    o_ref[...] = h * ns


_tmid = pl.pallas_call(
    _tmid_body,
    grid=(_G,),
    in_specs=[
        pl.BlockSpec((_NC, _R, _D), lambda i: (0, i, 0)),
        pl.BlockSpec((_R, 1), lambda i: (i, 0)),
        pl.BlockSpec((_R, 1), lambda i: (i, 0)),
        pl.BlockSpec((_R, 1), lambda i: (i, 0)),
        pl.BlockSpec((_R, 1), lambda i: (i, 0)),
        pl.BlockSpec((1, _D), lambda i: (0, 0)),
        pl.BlockSpec((1, _D), lambda i: (0, 0)),
        pl.BlockSpec((_D, _D), lambda i: (0, 0)),
    ],
    out_specs=pl.BlockSpec((_R, _D), lambda i: (i, 0)),
    out_shape=jax.ShapeDtypeStruct((_N, _D), jnp.float32),
)


def _t3_body(agg_ref, d0_ref, d1_ref, b_ref, w_ref, o_ref):
    h = jnp.dot(agg_ref[0] + agg_ref[1], w_ref[...],
                preferred_element_type=jnp.float32)
    nd = lax.rsqrt(jnp.maximum(d0_ref[...] + d1_ref[...], 1.0))
    o_ref[...] = h * nd + b_ref[...]


_t3 = pl.pallas_call(
    _t3_body,
    grid=(_G,),
    in_specs=[
        pl.BlockSpec((_NC, _R, _D), lambda i: (0, i, 0)),
        pl.BlockSpec((_R, 1), lambda i: (i, 0)),
        pl.BlockSpec((_R, 1), lambda i: (i, 0)),
        pl.BlockSpec((1, _D), lambda i: (0, 0)),
        pl.BlockSpec((_D, _D), lambda i: (0, 0)),
    ],
    out_specs=pl.BlockSpec((_R, _D), lambda i: (i, 0)),
    out_shape=jax.ShapeDtypeStruct((_N, _D), jnp.float32),
)


def kernel(feat, edge_index, W1, b1, a1, W2, b2, a2, W3, b3):
    eflat = edge_index.reshape(2 * _E)
    degS, degD = _degrees(eflat)
    s0 = degS[0].reshape(_NP, 1)
    s1 = degS[1].reshape(_NP, 1)
    d0 = degD[0].reshape(_NP, 1)
    d1 = degD[1].reshape(_NP, 1)
    zeros = jnp.zeros((_NP, _D), jnp.float32)
    b1r, a1r = b1.reshape(1, _D), a1.reshape(1, _D)
    b2r, a2r = b2.reshape(1, _D), a2.reshape(1, _D)
    b3r = b3.reshape(1, _D)

    q = _t0(feat, s0, s1)
    agg = _aggregate(q, eflat, zeros)
    q = _tmid(agg, d0, d1, s0, s1, b1r, a1r, W1)
    agg = _aggregate(q, eflat, zeros)
    q = _tmid(agg, d0, d1, s0, s1, b2r, a2r, W2)
    agg = _aggregate(q, eflat, zeros)
    return _t3(agg, d0, d1, b3r, W3)


# speedup vs baseline: 1.3431x; 1.0100x over previous
import functools

import jax
import jax.numpy as jnp
from jax import lax
from jax.experimental import pallas as pl
from jax.experimental.pallas import tpu as pltpu
from jax.experimental.pallas import tpu_sc as plsc

_N = 10000
_E = 320000
_D = 128
_NP = 10240
_NC, _NS = 2, 16
_NW = _NC * _NS
_EW = _E // _NW
_CH = 128
_NF = _EW // _CH
_TAIL = _EW - _NF * _CH
_NBUF = 2
_RPS = _NP // _NS
_DPS = _NP // _NS

_mesh = plsc.VectorSubcoreMesh(core_axis_name="c", subcore_axis_name="s")


@functools.partial(
    pl.kernel,
    out_type=(jax.ShapeDtypeStruct((_NC, _NP), jnp.float32),
              jax.ShapeDtypeStruct((_NC, _NP), jnp.float32)),
    mesh=_mesh,
    scratch_types=(
        pltpu.VMEM((_EW,), jnp.int32),
        pltpu.VMEM((_EW,), jnp.int32),
        tuple(pltpu.VMEM((_CH,), jnp.int32) for _ in range(2)),
        tuple(pltpu.VMEM((_CH,), jnp.int32) for _ in range(2)),
        pltpu.VMEM((_TAIL,), jnp.int32),
        pltpu.VMEM((_TAIL,), jnp.int32),
        tuple(pltpu.SemaphoreType.DMA for _ in range(2)),
        tuple(pltpu.SemaphoreType.DMA for _ in range(2)),
        pltpu.VMEM((_CH,), jnp.float32),
        pltpu.VMEM((_DPS,), jnp.float32),
        pltpu.VMEM_SHARED((_NP,), jnp.float32),
        pltpu.VMEM_SHARED((_NP,), jnp.float32),
    ),
)
def _degrees(edge_hbm, outs_hbm, outd_hbm,
             sidx, didx, sidx_b, didx_b, sidx_t, didx_t, ssems, dsems,
             ones_v, zer_v, accs, accd):
    cid = lax.axis_index("c")
    sid = lax.axis_index("s")
    w = sid * _NC + cid
    one = jnp.full((16,), 1.0, jnp.float32)
    zero = jnp.zeros((16,), jnp.float32)
    for j in range(_CH // 16):
        ones_v[pl.ds(16 * j, 16)] = one
    for j in range(_DPS // 16):
        zer_v[pl.ds(16 * j, 16)] = zero
    base = sid * _DPS
    pltpu.sync_copy(zer_v, accs.at[pl.ds(base, _DPS)])
    pltpu.sync_copy(zer_v, accd.at[pl.ds(base, _DPS)])
    pltpu.sync_copy(edge_hbm.at[pl.ds(w * _EW, _EW)], sidx)
    pltpu.sync_copy(edge_hbm.at[pl.ds(_E + w * _EW, _EW)], didx)
    plsc.subcore_barrier()

    def fill_and_scat(j, p):
        for v in range(_CH // 16):
            sl = pl.ds(j * _CH + 16 * v, 16)
            sidx_b[p][pl.ds(16 * v, 16)] = sidx[sl]
            didx_b[p][pl.ds(16 * v, 16)] = didx[sl]
        pltpu.async_copy(ones_v, accs.at[sidx_b[p]], ssems[p], add=True)
        pltpu.async_copy(ones_v, accd.at[didx_b[p]], dsems[p], add=True)

    def ddrain(p):
        pltpu.make_async_copy(ones_v, accs.at[sidx_b[p]], ssems[p]).wait()
        pltpu.make_async_copy(ones_v, accd.at[didx_b[p]], dsems[p]).wait()

    for p in range(2):
        fill_and_scat(p, p)

    def body(m, carry):
        for p in range(2):
            ddrain(p)
            fill_and_scat(2 * m + 2 + p, p)
        return carry

    lax.fori_loop(0, _NF // 2 - 1, body, 0)
    for p in range(2):
        ddrain(p)

    sidx_t[...] = sidx[pl.ds(_NF * _CH, _TAIL)]
    didx_t[...] = didx[pl.ds(_NF * _CH, _TAIL)]
    pltpu.sync_copy(ones_v.at[pl.ds(0, _TAIL)], accs.at[sidx_t], add=True)
    pltpu.sync_copy(ones_v.at[pl.ds(0, _TAIL)], accd.at[didx_t], add=True)

    plsc.subcore_barrier()
    pltpu.sync_copy(accs.at[pl.ds(base, _DPS)],
                    outs_hbm.at[cid, pl.ds(base, _DPS)])
    pltpu.sync_copy(accd.at[pl.ds(base, _DPS)],
                    outd_hbm.at[cid, pl.ds(base, _DPS)])


@functools.partial(
    pl.kernel,
    out_type=jax.ShapeDtypeStruct((_NC, _NP, _D), jnp.float32),
    mesh=_mesh,
    scratch_types=(
        pltpu.VMEM((_EW,), jnp.int32),
        tuple(pltpu.VMEM((_CH,), jnp.int32) for _ in range(_NBUF)),
        pltpu.VMEM((_TAIL,), jnp.int32),
        tuple(pltpu.VMEM((_CH, _D), jnp.float32) for _ in range(_NBUF)),
        tuple(pltpu.SemaphoreType.DMA for _ in range(_NBUF)),
        tuple(pltpu.SemaphoreType.DMA for _ in range(_NBUF)),
        pltpu.SemaphoreType.DMA,
        pltpu.SemaphoreType.DMA,
        pltpu.VMEM_SHARED((_NP, _D), jnp.float32),
    ),
)
def _aggregate(h_hbm, edge_hbm, zeros_hbm, out_hbm,
               sidx, didx_b, didx_t, rows, gsems, isems, lsem, zsem, acc):
    cid = lax.axis_index("c")
    sid = lax.axis_index("s")
    w = sid * _NC + cid
    ebase = w * _EW
    rowbase = sid * _RPS

    zdesc = pltpu.async_copy(zeros_hbm,
                             acc.at[pl.ds(rowbase, _RPS)], zsem)
    ldesc = pltpu.async_copy(edge_hbm.at[pl.ds(ebase, _EW)], sidx, lsem)
    ldesc.wait()

    def start(k, b):
        pltpu.async_copy(edge_hbm.at[pl.ds(_E + ebase + k * _CH, _CH)],
                         didx_b[b], isems[b])
        pltpu.async_copy(h_hbm.at[sidx.at[pl.ds(k * _CH, _CH)]],
                         rows[b], gsems[b])

    def finish(k, b):
        pltpu.make_async_copy(edge_hbm.at[pl.ds(_E + ebase + k * _CH, _CH)],
                              didx_b[b], isems[b]).wait()
        pltpu.make_async_copy(h_hbm.at[sidx.at[pl.ds(k * _CH, _CH)]],
                              rows[b], gsems[b]).wait()
        pltpu.sync_copy(rows[b], acc.at[didx_b[b]], add=True)

    for b in range(_NBUF):
        start(b, b)

    zdesc.wait()
    plsc.subcore_barrier()

    def body(j, carry):
        for b in range(_NBUF):
            k = _NBUF * j + b
            finish(k, b)
            start(k + _NBUF, b)
        return carry

    lax.fori_loop(0, _NF // _NBUF - 1, body, 0)
    for b in range(_NBUF):
        finish(_NF - _NBUF + b, b)

    pltpu.sync_copy(edge_hbm.at[pl.ds(_E + ebase + _NF * _CH, _TAIL)], didx_t)
    pltpu.sync_copy(h_hbm.at[sidx.at[pl.ds(_NF * _CH, _TAIL)]],
                    rows[0].at[pl.ds(0, _TAIL)])
    pltpu.sync_copy(rows[0].at[pl.ds(0, _TAIL)], acc.at[didx_t], add=True)

    plsc.subcore_barrier()
    pltpu.sync_copy(acc.at[pl.ds(rowbase, _RPS)],
                    out_hbm.at[cid, pl.ds(rowbase, _RPS)])


_R = 2000
_G = _N // _R


def _t0_body(x_ref, g_ref, o_ref):
    g = g_ref[...]
    ns = lax.rsqrt(jnp.maximum(g[:, 0:1] + g[:, 1:2], 1.0))
    o_ref[...] = x_ref[...] * ns


_t0 = pl.pallas_call(
    _t0_body,
    grid=(_G,),
    in_specs=[
        pl.BlockSpec((_R, _D), lambda i: (i, 0)),
        pl.BlockSpec((_R, 4), lambda i: (i, 0)),
    ],
    out_specs=pl.BlockSpec((_R, _D), lambda i: (i, 0)),
    out_shape=jax.ShapeDtypeStruct((_N, _D), jnp.float32),
)


def _tmid_body(agg_ref, g_ref, b_ref, a_ref, w_ref, o_ref):
    h = jnp.dot(agg_ref[0] + agg_ref[1], w_ref[...],
                preferred_element_type=jnp.float32)
    g = g_ref[...]
    nd = lax.rsqrt(jnp.maximum(g[:, 2:3] + g[:, 3:4], 1.0))
    h = h * nd + b_ref[...]
    h = jnp.where(h >= 0, h, a_ref[...] * h)
    ns = lax.rsqrt(jnp.maximum(g[:, 0:1] + g[:, 1:2], 1.0))
    o_ref[...] = h * ns


_tmid = pl.pallas_call(
    _tmid_body,
    grid=(_G,),
    in_specs=[
        pl.BlockSpec((_NC, _R, _D), lambda i: (0, i, 0)),
        pl.BlockSpec((_R, 4), lambda i: (i, 0)),
        pl.BlockSpec((1, _D), lambda i: (0, 0)),
        pl.BlockSpec((1, _D), lambda i: (0, 0)),
        pl.BlockSpec((_D, _D), lambda i: (0, 0)),
    ],
    out_specs=pl.BlockSpec((_R, _D), lambda i: (i, 0)),
    out_shape=jax.ShapeDtypeStruct((_N, _D), jnp.float32),
)


def _t3_body(agg_ref, g_ref, b_ref, w_ref, o_ref):
    h = jnp.dot(agg_ref[0] + agg_ref[1], w_ref[...],
                preferred_element_type=jnp.float32)
    g = g_ref[...]
    nd = lax.rsqrt(jnp.maximum(g[:, 2:3] + g[:, 3:4], 1.0))
    o_ref[...] = h * nd + b_ref[...]


_t3 = pl.pallas_call(
    _t3_body,
    grid=(_G,),
    in_specs=[
        pl.BlockSpec((_NC, _R, _D), lambda i: (0, i, 0)),
        pl.BlockSpec((_R, 4), lambda i: (i, 0)),
        pl.BlockSpec((1, _D), lambda i: (0, 0)),
        pl.BlockSpec((_D, _D), lambda i: (0, 0)),
    ],
    out_specs=pl.BlockSpec((_R, _D), lambda i: (i, 0)),
    out_shape=jax.ShapeDtypeStruct((_N, _D), jnp.float32),
)


def kernel(feat, edge_index, W1, b1, a1, W2, b2, a2, W3, b3):
    eflat = edge_index.reshape(2 * _E)
    degS, degD = _degrees(eflat)
    g4 = jnp.stack([degS[0], degS[1], degD[0], degD[1]], axis=1)
    zeros = jnp.zeros((_RPS, _D), jnp.float32)
    b1r, a1r = b1.reshape(1, _D), a1.reshape(1, _D)
    b2r, a2r = b2.reshape(1, _D), a2.reshape(1, _D)
    b3r = b3.reshape(1, _D)

    q = _t0(feat, g4)
    agg = _aggregate(q, eflat, zeros)
    q = _tmid(agg, g4, b1r, a1r, W1)
    agg = _aggregate(q, eflat, zeros)
    q = _tmid(agg, g4, b2r, a2r, W2)
    agg = _aggregate(q, eflat, zeros)
    return _t3(agg, g4, b3r, W3)
